# baseline (speedup 1.0000x reference)
import math

import jax
import jax.numpy as jnp
from jax import lax
from jax.experimental import pallas as pl
from jax.experimental.pallas import tpu as pltpu
from jax.experimental.pallas import tpu_sc as plsc

N = 10000
E = 320000
H = 128
NPAD = 10240
BN = 1000
NC = 2
NS = 16
CHUNK = 80
EPT = E // NS
NCHUNK = EPT // CHUNK
RPT = NPAD // NS
NBUF = 4
LOOK = 1

_INV_SQRT2 = 1.0 / math.sqrt(2.0)


def _gelu(x):
    return 0.5 * x * (1.0 + lax.erf(x * _INV_SQRT2))


def _dot(a, b):
    return jnp.dot(a, b, preferred_element_type=jnp.float32)


def _prep_body(x_ref, pos_ref, emb_ref, w1a_ref, w1b_ref, b1_ref, w2_ref,
               b2_ref, w_ref, bw_ref, wt_ref, bwt_ref,
               h_ref, u_ref, v_ref):
    x = x_ref[0]
    oht = (x == lax.broadcasted_iota(jnp.int32, (16, BN), 0)).astype(jnp.float32)
    hx = lax.dot_general(oht, emb_ref[...], (((0,), (0,)), ((), ())),
                         preferred_element_type=jnp.float32)
    pos = pos_ref[...].astype(jnp.float32)
    k = lax.broadcasted_iota(jnp.int32, (1, 32), 1).astype(jnp.float32)
    dt = jnp.exp(k * (-2.0 * math.log(10000.0) / 64.0))
    ang = pos * dt
    pe = jnp.concatenate([jnp.sin(ang), jnp.cos(ang)], axis=-1)
    t = _gelu(hx @ w1a_ref[...] + pe @ w1b_ref[...] + b1_ref[...])
    h = _dot(t, w2_ref[...]) + b2_ref[...]
    h_ref[...] = h
    u_ref[...] = _dot(h, w_ref[...]) + bw_ref[...]
    v_ref[...] = _dot(h, wt_ref[...]) + bwt_ref[...]


def _self_body(h_ref, ws_ref, bws_ref, s_ref):
    s_ref[...] = _dot(h_ref[...], ws_ref[...]) + bws_ref[...]


def _side2_body(h0_ref, h1_ref, ws_ref, bws_ref, oa_ref, ob_ref, ob1_ref,
                s_ref, p_ref):
    h1 = h1_ref[...]
    s_ref[...] = _dot(h1, ws_ref[...]) + bws_ref[...]
    p_ref[...] = (_dot(h0_ref[...], oa_ref[...]) + _dot(h1, ob_ref[...])
                  + ob1_ref[...])


def _combine_body(s_ref, mf_ref, mb_ref, w_ref, bw_ref, wt_ref, bwt_ref,
                  h1_ref, u_ref, v_ref):
    h1 = _gelu(mf_ref[...] + mb_ref[...] + s_ref[...])
    h1_ref[...] = h1
    u_ref[...] = _dot(h1, w_ref[...]) + bw_ref[...]
    v_ref[...] = _dot(h1, wt_ref[...]) + bwt_ref[...]


def _final_body(s_ref, mf_ref, mb_ref, p_ref, oc_ref, ow2_ref, ob2_ref,
                out_ref):
    h2 = _gelu(mf_ref[...] + mb_ref[...] + s_ref[...])
    t = _gelu(p_ref[...] + _dot(h2, oc_ref[...]))
    out_ref[...] = _dot(t, ow2_ref[...]) + ob2_ref[...]


def _row_spec(w):
    return pl.BlockSpec((BN, w), lambda i: (i, 0))


def _full_spec(shape):
    return pl.BlockSpec(shape, lambda i: (0,) * len(shape))


def _tc_call(body, in_specs, n_out):
    return pl.pallas_call(
        body,
        grid=(N // BN,),
        in_specs=in_specs,
        out_specs=[_row_spec(H)] * n_out,
        out_shape=[jax.ShapeDtypeStruct((N, H), jnp.float32)] * n_out,
    )


def _spmm_body(u_ref, v_ref, src_ref, dst_ref, z_ref, mf_ref, mb_ref, *scr):
    gidxb = scr[0:NBUF]
    sidxb = scr[NBUF:2 * NBUF]
    rows = scr[2 * NBUF:3 * NBUF]
    acc = scr[3 * NBUF]
    isem = scr[3 * NBUF + 1:4 * NBUF + 1]
    gsem = scr[4 * NBUF + 1:5 * NBUF + 1]
    ssem = scr[5 * NBUF + 1:6 * NBUF + 1]
    psem = scr[6 * NBUF + 1]
    cid = lax.axis_index("c")
    sid = lax.axis_index("s")
    r0 = sid * RPT
    zrow = z_ref.at[pl.ds(0, CHUNK)]

    def run(table, g_hbm, s_hbm, out_ref):
        e0 = sid * EPT

        def idx_fetch(g, slot):
            off = e0 + g * CHUNK
            pltpu.async_copy(g_hbm.at[pl.ds(off, CHUNK)], gidxb[slot],
                             isem[slot])
            pltpu.async_copy(s_hbm.at[pl.ds(off, CHUNK)], sidxb[slot],
                             isem[slot])

        def idx_wait(slot):
            pltpu.make_async_copy(g_hbm.at[pl.ds(0, CHUNK)], gidxb[slot],
                                  isem[slot]).wait()
            pltpu.make_async_copy(s_hbm.at[pl.ds(0, CHUNK)], sidxb[slot],
                                  isem[slot]).wait()

        zcp = pltpu.async_copy(z_ref.at[pl.ds(r0, RPT)],
                               acc.at[pl.ds(r0, RPT)], psem)
        for c in range(LOOK + 1):
            idx_fetch(c, c)
        for c in range(LOOK):
            idx_wait(c)
            pltpu.async_copy(table.at[gidxb[c]], rows[c], gsem[c])
        zcp.wait()
        plsc.subcore_barrier()

        def superstep(t, carry):
            for b in range(NBUF):
                g = t * NBUF + b
                pltpu.make_async_copy(zrow, rows[b], gsem[b]).wait()
                pltpu.async_copy(rows[b], acc.at[sidxb[b]], ssem[b],
                                 add=True)
                ga = g + LOOK + 1
                ba = (b + LOOK + 1) % NBUF
                @pl.when(jnp.logical_and(ga >= NBUF, ga < NCHUNK))
                def _():
                    pltpu.make_async_copy(zrow, rows[ba], ssem[ba]).wait()

                @pl.when(ga < NCHUNK)
                def _():
                    idx_fetch(ga, ba)

                gb = g + LOOK
                bb = (b + LOOK) % NBUF
                idx_wait(bb)
                pltpu.async_copy(table.at[gidxb[bb]], rows[bb], gsem[bb])
            return carry

        lax.fori_loop(0, NCHUNK // NBUF, superstep, 0)
        tail0 = NCHUNK - NCHUNK % NBUF
        for g in range(tail0, NCHUNK):
            if g >= tail0 + LOOK:
                idx_wait(g % NBUF)
                pltpu.async_copy(table.at[gidxb[g % NBUF]], rows[g % NBUF],
                                 gsem[g % NBUF])
        for g in range(tail0, NCHUNK):
            b = g % NBUF
            pltpu.make_async_copy(zrow, rows[b], gsem[b]).wait()
            pltpu.async_copy(rows[b], acc.at[sidxb[b]], ssem[b], add=True)
        for b in range(NBUF):
            pltpu.make_async_copy(zrow, rows[b], ssem[b]).wait()
        plsc.subcore_barrier()
        pltpu.sync_copy(acc.at[pl.ds(r0, RPT)], out_ref.at[pl.ds(r0, RPT)])

    @pl.when(cid == 0)
    def _():
        run(u_ref, src_ref, dst_ref, mf_ref)

    @pl.when(cid == 1)
    def _():
        run(v_ref, dst_ref, src_ref, mb_ref)


def _spmm(u, v, src1, dst1, zeros):
    mesh = plsc.VectorSubcoreMesh(core_axis_name="c", subcore_axis_name="s",
                                  num_cores=NC, num_subcores=NS)
    f = pl.kernel(
        _spmm_body,
        out_type=(jax.ShapeDtypeStruct((NPAD, H), jnp.float32),) * 2,
        mesh=mesh,
        scratch_types=(
            [pltpu.VMEM((CHUNK,), jnp.int32)] * (2 * NBUF)
            + [pltpu.VMEM((CHUNK, H), jnp.float32)] * NBUF
            + [pltpu.VMEM_SHARED((NPAD, H), jnp.float32)]
            + [pltpu.SemaphoreType.DMA] * (3 * NBUF + 1)
        ),
    )
    return f(u, v, src1, dst1, zeros)


def kernel(x_n, edge_index, abs_level, rel_level, emb0, p_w1, p_b1, p_w2,
           p_b2, l0_W, l0_bW, l0_Wt, l0_bWt, l0_Ws, l0_bWs,
           l1_W, l1_bW, l1_Wt, l1_bWt, l1_Ws, l1_bWs,
           o_w1, o_b1, o_w2, o_b2):
    del rel_level
    xp = x_n.reshape(N // BN, 1, BN)
    src = edge_index[0]
    dst = edge_index[1]
    zeros = jnp.zeros((NPAD, H), jnp.float32)
    r1 = lambda b: b.reshape(1, H)

    h0, u0, v0 = _tc_call(
        _prep_body,
        [pl.BlockSpec((1, 1, BN), lambda i: (i, 0, 0)), _row_spec(1),
         _full_spec((16, H)), _full_spec((H, H)),
         _full_spec((64, H)), _full_spec((1, H)), _full_spec((H, H)),
         _full_spec((1, H)), _full_spec((H, H)), _full_spec((1, H)),
         _full_spec((H, H)), _full_spec((1, H))],
        3,
    )(xp, abs_level, emb0, p_w1[:H], p_w1[H:], r1(p_b1), p_w2, r1(p_b2),
      l0_W, r1(l0_bW), l0_Wt, r1(l0_bWt))

    mf0, mb0 = _spmm(u0, v0, src, dst, zeros)

    (s0,) = _tc_call(
        _self_body,
        [_row_spec(H), _full_spec((H, H)), _full_spec((1, H))],
        1,
    )(h0, l0_Ws, r1(l0_bWs))

    h1, u1, v1 = _tc_call(
        _combine_body,
        [_row_spec(H)] * 3 + [_full_spec((H, H)), _full_spec((1, H)),
         _full_spec((H, H)), _full_spec((1, H))],
        3,
    )(s0, mf0, mb0, l1_W, r1(l1_bW), l1_Wt, r1(l1_bWt))

    mf1, mb1 = _spmm(u1, v1, src, dst, zeros)

    s1, p01 = _tc_call(
        _side2_body,
        [_row_spec(H)] * 2 + [_full_spec((H, H)), _full_spec((1, H)),
         _full_spec((H, H)), _full_spec((H, H)), _full_spec((1, H))],
        2,
    )(h0, h1, l1_Ws, r1(l1_bWs), o_w1[:H], o_w1[H:2 * H], r1(o_b1))

    (out,) = _tc_call(
        _final_body,
        [_row_spec(H)] * 4 + [_full_spec((H, H)), _full_spec((H, H)),
         _full_spec((1, H))],
        1,
    )(s1, mf1, mb1, p01, o_w1[2 * H:], o_w2, r1(o_b2))

    return out


# speedup vs baseline: 1.5370x; 1.5370x over previous
import math

import jax
import jax.numpy as jnp
from jax import lax
from jax.experimental import pallas as pl
from jax.experimental.pallas import tpu as pltpu
from jax.experimental.pallas import tpu_sc as plsc

N = 10000
E = 320000
H = 128
NPAD = 10240
BN = 1000
NC = 2
NS = 16
CHUNK = 40
EPT = E // NS
NCHUNK = EPT // CHUNK
RPT = NPAD // NS
NBUF = 8
LOOK = 6

_INV_SQRT2 = 1.0 / math.sqrt(2.0)


def _gelu(x):
    return 0.5 * x * (1.0 + lax.erf(x * _INV_SQRT2))


def _dot(a, b):
    return jnp.dot(a, b, preferred_element_type=jnp.float32)


def _prep_body(x_ref, pos_ref, emb_ref, w1a_ref, w1b_ref, b1_ref, w2_ref,
               b2_ref, w_ref, bw_ref, wt_ref, bwt_ref,
               h_ref, u_ref, v_ref):
    x = x_ref[0]
    oht = (x == lax.broadcasted_iota(jnp.int32, (16, BN), 0)).astype(jnp.float32)
    hx = lax.dot_general(oht, emb_ref[...], (((0,), (0,)), ((), ())),
                         preferred_element_type=jnp.float32)
    pos = pos_ref[...].astype(jnp.float32)
    k = lax.broadcasted_iota(jnp.int32, (1, 32), 1).astype(jnp.float32)
    dt = jnp.exp(k * (-2.0 * math.log(10000.0) / 64.0))
    ang = pos * dt
    pe = jnp.concatenate([jnp.sin(ang), jnp.cos(ang)], axis=-1)
    t = _gelu(hx @ w1a_ref[...] + pe @ w1b_ref[...] + b1_ref[...])
    h = _dot(t, w2_ref[...]) + b2_ref[...]
    h_ref[...] = h
    u_ref[...] = _dot(h, w_ref[...]) + bw_ref[...]
    v_ref[...] = _dot(h, wt_ref[...]) + bwt_ref[...]


def _self_body(h_ref, ws_ref, bws_ref, s_ref):
    s_ref[...] = _dot(h_ref[...], ws_ref[...]) + bws_ref[...]


def _side2_body(h0_ref, h1_ref, ws_ref, bws_ref, oa_ref, ob_ref, ob1_ref,
                s_ref, p_ref):
    h1 = h1_ref[...]
    s_ref[...] = _dot(h1, ws_ref[...]) + bws_ref[...]
    p_ref[...] = (_dot(h0_ref[...], oa_ref[...]) + _dot(h1, ob_ref[...])
                  + ob1_ref[...])


def _combine_body(s_ref, mf_ref, mb_ref, w_ref, bw_ref, wt_ref, bwt_ref,
                  h1_ref, u_ref, v_ref):
    h1 = _gelu(mf_ref[...] + mb_ref[...] + s_ref[...])
    h1_ref[...] = h1
    u_ref[...] = _dot(h1, w_ref[...]) + bw_ref[...]
    v_ref[...] = _dot(h1, wt_ref[...]) + bwt_ref[...]


def _final_body(s_ref, mf_ref, mb_ref, p_ref, oc_ref, ow2_ref, ob2_ref,
                out_ref):
    h2 = _gelu(mf_ref[...] + mb_ref[...] + s_ref[...])
    t = _gelu(p_ref[...] + _dot(h2, oc_ref[...]))
    out_ref[...] = _dot(t, ow2_ref[...]) + ob2_ref[...]


def _row_spec(w):
    return pl.BlockSpec((BN, w), lambda i: (i, 0))


def _full_spec(shape):
    return pl.BlockSpec(shape, lambda i: (0,) * len(shape))


def _tc_call(body, in_specs, n_out):
    return pl.pallas_call(
        body,
        grid=(N // BN,),
        in_specs=in_specs,
        out_specs=[_row_spec(H)] * n_out,
        out_shape=[jax.ShapeDtypeStruct((N, H), jnp.float32)] * n_out,
    )


def _spmm_body(u_ref, v_ref, src_ref, dst_ref, z_ref, mf_ref, mb_ref, *scr):
    gidxb = scr[0:NBUF]
    sidxb = scr[NBUF:2 * NBUF]
    rows = scr[2 * NBUF:3 * NBUF]
    acc = scr[3 * NBUF]
    isem = scr[3 * NBUF + 1:4 * NBUF + 1]
    gsem = scr[4 * NBUF + 1:5 * NBUF + 1]
    ssem = scr[5 * NBUF + 1:6 * NBUF + 1]
    psem = scr[6 * NBUF + 1]
    cid = lax.axis_index("c")
    sid = lax.axis_index("s")
    r0 = sid * RPT
    zrow = z_ref.at[pl.ds(0, CHUNK)]

    def run(table, g_hbm, s_hbm, out_ref):
        e0 = sid * EPT

        def idx_fetch(g, slot):
            off = e0 + g * CHUNK
            pltpu.async_copy(g_hbm.at[pl.ds(off, CHUNK)], gidxb[slot],
                             isem[slot])
            pltpu.async_copy(s_hbm.at[pl.ds(off, CHUNK)], sidxb[slot],
                             isem[slot])

        def idx_wait(slot):
            pltpu.make_async_copy(g_hbm.at[pl.ds(0, CHUNK)], gidxb[slot],
                                  isem[slot]).wait()
            pltpu.make_async_copy(s_hbm.at[pl.ds(0, CHUNK)], sidxb[slot],
                                  isem[slot]).wait()

        zcp = pltpu.async_copy(z_ref.at[pl.ds(r0, RPT)],
                               acc.at[pl.ds(r0, RPT)], psem)
        for c in range(LOOK + 1):
            idx_fetch(c, c)
        for c in range(LOOK):
            idx_wait(c)
            pltpu.async_copy(table.at[gidxb[c]], rows[c], gsem[c])
        zcp.wait()
        plsc.subcore_barrier()

        def superstep(t, carry):
            for b in range(NBUF):
                g = t * NBUF + b
                pltpu.make_async_copy(zrow, rows[b], gsem[b]).wait()
                pltpu.async_copy(rows[b], acc.at[sidxb[b]], ssem[b],
                                 add=True)
                ga = g + LOOK + 1
                ba = (b + LOOK + 1) % NBUF
                @pl.when(jnp.logical_and(ga >= NBUF, ga < NCHUNK))
                def _():
                    pltpu.make_async_copy(zrow, rows[ba], ssem[ba]).wait()

                @pl.when(ga < NCHUNK)
                def _():
                    idx_fetch(ga, ba)

                gb = g + LOOK
                bb = (b + LOOK) % NBUF
                @pl.when(gb < NCHUNK)
                def _():
                    idx_wait(bb)
                    pltpu.async_copy(table.at[gidxb[bb]], rows[bb], gsem[bb])
            return carry

        lax.fori_loop(0, NCHUNK // NBUF, superstep, 0)
        tail0 = NCHUNK - NCHUNK % NBUF
        for g in range(tail0, NCHUNK):
            if g >= tail0 + LOOK:
                idx_wait(g % NBUF)
                pltpu.async_copy(table.at[gidxb[g % NBUF]], rows[g % NBUF],
                                 gsem[g % NBUF])
        for g in range(tail0, NCHUNK):
            b = g % NBUF
            pltpu.make_async_copy(zrow, rows[b], gsem[b]).wait()
            pltpu.async_copy(rows[b], acc.at[sidxb[b]], ssem[b], add=True)
        for b in range(NBUF):
            pltpu.make_async_copy(zrow, rows[b], ssem[b]).wait()
        plsc.subcore_barrier()
        pltpu.sync_copy(acc.at[pl.ds(r0, RPT)], out_ref.at[pl.ds(r0, RPT)])

    @pl.when(cid == 0)
    def _():
        run(u_ref, src_ref, dst_ref, mf_ref)

    @pl.when(cid == 1)
    def _():
        run(v_ref, dst_ref, src_ref, mb_ref)


def _spmm(u, v, src1, dst1, zeros):
    mesh = plsc.VectorSubcoreMesh(core_axis_name="c", subcore_axis_name="s",
                                  num_cores=NC, num_subcores=NS)
    f = pl.kernel(
        _spmm_body,
        out_type=(jax.ShapeDtypeStruct((NPAD, H), jnp.float32),) * 2,
        mesh=mesh,
        scratch_types=(
            [pltpu.VMEM((CHUNK,), jnp.int32)] * (2 * NBUF)
            + [pltpu.VMEM((CHUNK, H), jnp.float32)] * NBUF
            + [pltpu.VMEM_SHARED((NPAD, H), jnp.float32)]
            + [pltpu.SemaphoreType.DMA] * (3 * NBUF + 1)
        ),
    )
    return f(u, v, src1, dst1, zeros)


def kernel(x_n, edge_index, abs_level, rel_level, emb0, p_w1, p_b1, p_w2,
           p_b2, l0_W, l0_bW, l0_Wt, l0_bWt, l0_Ws, l0_bWs,
           l1_W, l1_bW, l1_Wt, l1_bWt, l1_Ws, l1_bWs,
           o_w1, o_b1, o_w2, o_b2):
    del rel_level
    xp = x_n.reshape(N // BN, 1, BN)
    src = edge_index[0]
    dst = edge_index[1]
    zeros = jnp.zeros((NPAD, H), jnp.float32)
    r1 = lambda b: b.reshape(1, H)

    h0, u0, v0 = _tc_call(
        _prep_body,
        [pl.BlockSpec((1, 1, BN), lambda i: (i, 0, 0)), _row_spec(1),
         _full_spec((16, H)), _full_spec((H, H)),
         _full_spec((64, H)), _full_spec((1, H)), _full_spec((H, H)),
         _full_spec((1, H)), _full_spec((H, H)), _full_spec((1, H)),
         _full_spec((H, H)), _full_spec((1, H))],
        3,
    )(xp, abs_level, emb0, p_w1[:H], p_w1[H:], r1(p_b1), p_w2, r1(p_b2),
      l0_W, r1(l0_bW), l0_Wt, r1(l0_bWt))

    mf0, mb0 = _spmm(u0, v0, src, dst, zeros)

    (s0,) = _tc_call(
        _self_body,
        [_row_spec(H), _full_spec((H, H)), _full_spec((1, H))],
        1,
    )(h0, l0_Ws, r1(l0_bWs))

    h1, u1, v1 = _tc_call(
        _combine_body,
        [_row_spec(H)] * 3 + [_full_spec((H, H)), _full_spec((1, H)),
         _full_spec((H, H)), _full_spec((1, H))],
        3,
    )(s0, mf0, mb0, l1_W, r1(l1_bW), l1_Wt, r1(l1_bWt))

    mf1, mb1 = _spmm(u1, v1, src, dst, zeros)

    s1, p01 = _tc_call(
        _side2_body,
        [_row_spec(H)] * 2 + [_full_spec((H, H)), _full_spec((1, H)),
         _full_spec((H, H)), _full_spec((H, H)), _full_spec((1, H))],
        2,
    )(h0, h1, l1_Ws, r1(l1_bWs), o_w1[:H], o_w1[H:2 * H], r1(o_b1))

    (out,) = _tc_call(
        _final_body,
        [_row_spec(H)] * 4 + [_full_spec((H, H)), _full_spec((H, H)),
         _full_spec((1, H))],
        1,
    )(s1, mf1, mb1, p01, o_w1[2 * H:], o_w2, r1(o_b2))

    return out
